# baseline (speedup 1.0000x reference)
import functools

import jax
import jax.numpy as jnp
from jax import lax
from jax.experimental import pallas as pl
from jax.experimental.pallas import tpu as pltpu
from jax.experimental.pallas import tpu_sc as plsc


@functools.lru_cache(maxsize=None)
def _build_gather(Bt, T, D):
    info = plsc.get_sparse_core_info()
    NC, NS = info.num_cores, info.num_subcores
    NW = NC * NS
    IT = 128
    DT = D // 8
    n_itiles = Bt // IT
    it_per_w = n_itiles // NW
    assert Bt % IT == 0 and n_itiles % NW == 0 and D % 8 == 0
    mesh = plsc.VectorSubcoreMesh(core_axis_name="c", subcore_axis_name="s")

    @functools.partial(
        pl.kernel,
        mesh=mesh,
        out_type=jax.ShapeDtypeStruct((T, DT, n_itiles, 8, IT), jnp.float32),
        scratch_types=[
            pltpu.VMEM((IT * T,), jnp.int32),
            pltpu.VMEM((2, IT), jnp.int32),
            pltpu.VMEM((2, IT, D), jnp.float32),
            pltpu.VMEM((2, DT, 8, IT), jnp.float32),
            pltpu.SemaphoreType.DMA((2,)),
            pltpu.SemaphoreType.DMA((2,)),
        ],
        compiler_params=pltpu.CompilerParams(
            use_tc_tiling_on_sc=False, needs_layout_passes=False
        ),
    )
    def gather_kernel(
        idx_hbm, table_hbm, out_hbm, ids_v, idxcol_v, rows_v, otile_v, gsem, ssem
    ):
        wid = lax.axis_index("s") * NC + lax.axis_index("c")
        iota = lax.iota(jnp.int32, 16)
        iota_t = iota * T

        def extract(t, p):
            for g in range(IT // 16):
                v = plsc.load_gather(ids_v, [iota_t + (g * 16 * T + t)])
                idxcol_v[p, pl.ds(g * 16, 16)] = v

        def fire_gather(p):
            pltpu.async_copy(table_hbm.at[idxcol_v.at[p]], rows_v.at[p], gsem.at[p])

        def wait_gather(p):
            pltpu.make_async_copy(
                table_hbm.at[idxcol_v.at[p]], rows_v.at[p], gsem.at[p]
            ).wait()

        row_vecs = [iota + g * 16 for g in range(IT // 16)]

        def transpose(p):
            for dt in range(DT):
                for ds in range(8):
                    col = jnp.full((16,), dt * 8 + ds, jnp.int32)
                    vals = [
                        plsc.load_gather(rows_v.at[p], [row_vecs[g], col])
                        for g in range(IT // 16)
                    ]
                    for g in range(IT // 16):
                        otile_v[p, dt, ds, pl.ds(g * 16, 16)] = vals[g]

        def fire_store(t, itile, p):
            for dt in range(DT):
                pltpu.async_copy(
                    otile_v.at[p, dt], out_hbm.at[t, dt, itile], ssem.at[p]
                )

        def wait_store(t, itile, p):
            for dt in range(DT):
                pltpu.make_async_copy(
                    otile_v.at[p, dt], out_hbm.at[t, dt, itile], ssem.at[p]
                ).wait()

        def per_itile(il, carry):
            itile = wid * it_per_w + il
            base = itile * IT * T
            pltpu.sync_copy(idx_hbm.at[pl.ds(base, IT * T)], ids_v)
            extract(0, 0)
            fire_gather(0)

            def per_t(t, c):
                p = lax.rem(t, 2)
                wait_gather(p)

                @pl.when(t + 1 < T)
                def _():
                    extract(t + 1, 1 - p)
                    fire_gather(1 - p)

                @pl.when(t >= 2)
                def _():
                    wait_store(t - 2, itile, p)

                transpose(p)
                fire_store(t, itile, p)
                return c

            lax.fori_loop(0, T, per_t, 0)
            wait_store(T - 2, itile, 0)
            wait_store(T - 1, itile, 1)
            return carry

        lax.fori_loop(0, it_per_w, per_itile, 0)

    return gather_kernel


def kernel(token_ids, weight):
    Bt, T = token_ids.shape
    V, D = weight.shape
    idx = token_ids.reshape(Bt * T).astype(jnp.int32)
    flat5d = _build_gather(Bt, T, D)(idx, weight)
    return flat5d.transpose(2, 4, 0, 1, 3).reshape(Bt, T, D)


# speedup vs baseline: 1.5913x; 1.5913x over previous
import functools

import jax
import jax.numpy as jnp
from jax import lax
from jax.experimental import pallas as pl
from jax.experimental.pallas import tpu as pltpu
from jax.experimental.pallas import tpu_sc as plsc


@functools.lru_cache(maxsize=None)
def _build_gather(Bt, T, D):
    info = plsc.get_sparse_core_info()
    NC, NS = info.num_cores, info.num_subcores
    NW = NC * NS
    IT = 128
    SB = 2 * IT
    DT = D // 8
    n_itiles = Bt // IT
    sb_per_w = n_itiles // (2 * NW)
    assert Bt % IT == 0 and n_itiles % (2 * NW) == 0 and D % 16 == 0
    mesh = plsc.VectorSubcoreMesh(core_axis_name="c", subcore_axis_name="s")

    @functools.partial(
        pl.kernel,
        mesh=mesh,
        out_type=jax.ShapeDtypeStruct((T, DT, n_itiles, 8, IT), jnp.float32),
        scratch_types=[
            pltpu.VMEM((SB * T,), jnp.int32),
            pltpu.VMEM((2, SB), jnp.int32),
            pltpu.VMEM((2, SB, D), jnp.float32),
            pltpu.VMEM((2, 2, DT, 8, IT), jnp.float32),
            pltpu.SemaphoreType.DMA((2,)),
            pltpu.SemaphoreType.DMA((2,)),
        ],
        compiler_params=pltpu.CompilerParams(
            use_tc_tiling_on_sc=False, needs_layout_passes=False
        ),
    )
    def gather_kernel(
        idx_hbm, table_hbm, out_hbm, ids_v, idxcol_v, rows_v, otile_v, gsem, ssem
    ):
        wid = lax.axis_index("s") * NC + lax.axis_index("c")
        iota = lax.iota(jnp.int32, 16)
        iota_t = iota * T
        row_vecs = [iota + jb * 16 for jb in range(SB // 16)]
        kbase = [(iota + k) & 15 for k in range(16)]

        def extract(t, p):
            for g in range(SB // 16):
                v = plsc.load_gather(ids_v, [iota_t + (g * 16 * T + t)])
                idxcol_v[p, pl.ds(g * 16, 16)] = v

        def fire_gathers(p):
            for h in range(2):
                pltpu.async_copy(
                    table_hbm.at[idxcol_v.at[p, pl.ds(h * IT, IT)]],
                    rows_v.at[p, pl.ds(h * IT, IT)],
                    gsem.at[p],
                )

        def wait_gathers(p):
            for h in range(2):
                pltpu.make_async_copy(
                    table_hbm.at[idxcol_v.at[p, pl.ds(h * IT, IT)]],
                    rows_v.at[p, pl.ds(h * IT, IT)],
                    gsem.at[p],
                ).wait()

        def transpose(p):
            src = rows_v.at[p]

            def per_cb(cb, c):
                for k in range(16):
                    dv = kbase[k] + cb * 16
                    dtv = dv >> 3
                    dsv = dv & 7
                    vals = [
                        plsc.load_gather(src, [row_vecs[jb], dv])
                        for jb in range(SB // 16)
                    ]
                    for jb in range(SB // 16):
                        h = jb // (IT // 16)
                        plsc.store_scatter(
                            otile_v.at[p, h],
                            [dtv, dsv, row_vecs[jb % (IT // 16)]],
                            vals[jb],
                        )
                return c

            lax.fori_loop(0, D // 16, per_cb, 0)

        def fire_store(t, itile0, p):
            for h in range(2):
                pltpu.async_copy(
                    otile_v.at[p, h], out_hbm.at[t, :, itile0 + h], ssem.at[p]
                )

        def wait_store(t, itile0, p):
            for h in range(2):
                pltpu.make_async_copy(
                    otile_v.at[p, h], out_hbm.at[t, :, itile0 + h], ssem.at[p]
                ).wait()

        def per_sb(sb, carry):
            itile0 = (wid * sb_per_w + sb) * 2
            base = itile0 * IT * T
            pltpu.sync_copy(idx_hbm.at[pl.ds(base, SB * T)], ids_v)
            extract(0, 0)
            fire_gathers(0)

            def per_t2(t2, c):
                for b in range(2):
                    t = t2 * 2 + b
                    wait_gathers(b)

                    @pl.when(t + 1 < T)
                    def _(t=t, b=b):
                        extract(t + 1, 1 - b)
                        fire_gathers(1 - b)

                    @pl.when(t >= 2)
                    def _(t=t, b=b):
                        wait_store(t - 2, itile0, b)

                    transpose(b)
                    fire_store(t, itile0, b)
                return c

            lax.fori_loop(0, T // 2, per_t2, 0)
            wait_store(T - 2, itile0, 0)
            wait_store(T - 1, itile0, 1)
            return carry

        lax.fori_loop(0, sb_per_w, per_sb, 0)

    return gather_kernel


def kernel(token_ids, weight):
    Bt, T = token_ids.shape
    V, D = weight.shape
    IT = 128
    DT = D // 8
    n_itiles = Bt // IT
    idx = token_ids.reshape(Bt * T).astype(jnp.int32)
    flat5 = _build_gather(Bt, T, D)(idx, weight)
    return flat5.transpose(2, 4, 0, 1, 3).reshape(Bt, T, D)


# speedup vs baseline: 2.2513x; 1.4147x over previous
import functools

import jax
import jax.numpy as jnp
from jax import lax
from jax.experimental import pallas as pl
from jax.experimental.pallas import tpu as pltpu
from jax.experimental.pallas import tpu_sc as plsc


@functools.lru_cache(maxsize=None)
def _build_detile(V, D):
    info = plsc.get_sparse_core_info()
    NC, NS = info.num_cores, info.num_subcores
    NW = NC * NS
    VT = 128
    n_full = V // VT
    rem = V % VT
    nj = n_full // NW
    n_extra = n_full % NW
    assert nj % 2 == 0 and rem % 16 == 0
    mesh = plsc.VectorSubcoreMesh(core_axis_name="c", subcore_axis_name="s")

    @functools.partial(
        pl.kernel,
        mesh=mesh,
        out_type=jax.ShapeDtypeStruct((V * D,), jnp.float32),
        scratch_types=[
            pltpu.VMEM((D, VT), jnp.float32),
            pltpu.VMEM((D, VT), jnp.float32),
            pltpu.VMEM((D * VT,), jnp.float32),
            pltpu.VMEM((D * VT,), jnp.float32),
            pltpu.SemaphoreType.DMA((2,)),
            pltpu.SemaphoreType.DMA((2,)),
        ],
        compiler_params=pltpu.CompilerParams(needs_layout_passes=False),
    )
    def detile_kernel(
        wt_hbm, rem_hbm, out_hbm, slab0, slab1, stage0, stage1, lsem, ssem
    ):
        wid = lax.axis_index("s") * NC + lax.axis_index("c")
        iota = lax.iota(jnp.int32, 16)
        row_vecs = [iota + vb * 16 for vb in range(VT // 16)]
        kbase = [(iota + k) & 15 for k in range(16)]
        slabs = (slab0, slab1)
        stages = (stage0, stage1)

        def vt_of(j):
            return j * NW + wid

        def fire_load(j, p):
            pltpu.async_copy(
                wt_hbm.at[:, pl.ds(vt_of(j) * VT, VT)], slabs[p], lsem.at[p]
            )

        def wait_load(j, p):
            pltpu.make_async_copy(
                wt_hbm.at[:, pl.ds(vt_of(j) * VT, VT)], slabs[p], lsem.at[p]
            ).wait()

        def transpose(p, n_vb):
            src = slabs[p]
            dst = stages[p]

            def per_db(db, c):
                for k in range(16):
                    dv = kbase[k] + db * 16
                    vals = [
                        plsc.load_gather(src, [dv, row_vecs[vb]])
                        for vb in range(n_vb)
                    ]
                    for vb in range(n_vb):
                        plsc.store_scatter(
                            dst, [row_vecs[vb] * D + dv], vals[vb]
                        )
                return c

            lax.fori_loop(0, D // 16, per_db, 0)

        def fire_store(j, p):
            pltpu.async_copy(
                stages[p], out_hbm.at[pl.ds(vt_of(j) * VT * D, VT * D)],
                ssem.at[p],
            )

        def wait_store(j, p):
            pltpu.make_async_copy(
                stages[p], out_hbm.at[pl.ds(vt_of(j) * VT * D, VT * D)],
                ssem.at[p],
            ).wait()

        fire_load(0, 0)
        fire_load(1, 1)

        def per_j2(j2, carry):
            for b in range(2):
                j = j2 * 2 + b
                wait_load(j, b)

                @pl.when(j >= 2)
                def _(j=j, b=b):
                    wait_store(j - 2, b)

                transpose(b, VT // 16)
                fire_store(j, b)

                @pl.when(j + 2 < nj)
                def _(j=j, b=b):
                    fire_load(j + 2, b)
            return carry

        lax.fori_loop(0, nj // 2, per_j2, 0)
        wait_store(nj - 2, 0)
        wait_store(nj - 1, 1)

        @pl.when(wid < n_extra)
        def _():
            vt = nj * NW + wid
            pltpu.sync_copy(wt_hbm.at[:, pl.ds(vt * VT, VT)], slab0)
            transpose(0, VT // 16)
            pltpu.sync_copy(stage0, out_hbm.at[pl.ds(vt * VT * D, VT * D)])

        if rem:

            @pl.when(wid == n_extra)
            def _():
                pltpu.sync_copy(rem_hbm, stage0.at[pl.ds(0, rem * D)])
                pltpu.sync_copy(
                    stage0.at[pl.ds(0, rem * D)],
                    out_hbm.at[pl.ds(n_full * VT * D, rem * D)],
                )

    return detile_kernel


@functools.lru_cache(maxsize=None)
def _build_gather(Bt, T, D):
    info = plsc.get_sparse_core_info()
    NC, NS = info.num_cores, info.num_subcores
    NW = NC * NS
    IT = 128
    SB = 2 * IT
    DT = D // 8
    n_itiles = Bt // IT
    sb_per_w = n_itiles // (2 * NW)
    assert Bt % IT == 0 and n_itiles % (2 * NW) == 0 and D % 16 == 0
    mesh = plsc.VectorSubcoreMesh(core_axis_name="c", subcore_axis_name="s")

    @functools.partial(
        pl.kernel,
        mesh=mesh,
        out_type=jax.ShapeDtypeStruct((T, DT, n_itiles, 8, IT), jnp.float32),
        scratch_types=[
            pltpu.VMEM((SB * T,), jnp.int32),
            pltpu.VMEM((2, SB), jnp.int32),
            pltpu.VMEM((2, SB, D), jnp.float32),
            pltpu.VMEM((2, 2, DT, 8, IT), jnp.float32),
            pltpu.SemaphoreType.DMA((2,)),
            pltpu.SemaphoreType.DMA((2,)),
        ],
        compiler_params=pltpu.CompilerParams(
            use_tc_tiling_on_sc=False, needs_layout_passes=False
        ),
    )
    def gather_kernel(
        idx_hbm, table_hbm, out_hbm, ids_v, idxcol_v, rows_v, otile_v, gsem, ssem
    ):
        wid = lax.axis_index("s") * NC + lax.axis_index("c")
        iota = lax.iota(jnp.int32, 16)
        iota_t = iota * T
        row_vecs = [iota + jb * 16 for jb in range(SB // 16)]
        kbase = [(iota + k) & 15 for k in range(16)]

        def extract(t, p):
            for g in range(SB // 16):
                v = plsc.load_gather(ids_v, [iota_t + (g * 16 * T + t)])
                idxcol_v[p, pl.ds(g * 16, 16)] = v

        def fire_gathers(p):
            for h in range(2):
                pltpu.async_copy(
                    table_hbm.at[idxcol_v.at[p, pl.ds(h * IT, IT)]],
                    rows_v.at[p, pl.ds(h * IT, IT)],
                    gsem.at[p],
                )

        def wait_gathers(p):
            for h in range(2):
                pltpu.make_async_copy(
                    table_hbm.at[idxcol_v.at[p, pl.ds(h * IT, IT)]],
                    rows_v.at[p, pl.ds(h * IT, IT)],
                    gsem.at[p],
                ).wait()

        def transpose(p):
            src = rows_v.at[p]

            def per_cb(cb, c):
                for k in range(16):
                    dv = kbase[k] + cb * 16
                    dtv = dv >> 3
                    dsv = dv & 7
                    vals = [
                        plsc.load_gather(src, [row_vecs[jb], dv])
                        for jb in range(SB // 16)
                    ]
                    for jb in range(SB // 16):
                        h = jb // (IT // 16)
                        plsc.store_scatter(
                            otile_v.at[p, h],
                            [dtv, dsv, row_vecs[jb % (IT // 16)]],
                            vals[jb],
                        )
                return c

            lax.fori_loop(0, D // 16, per_cb, 0)

        def fire_store(t, itile0, p):
            for h in range(2):
                pltpu.async_copy(
                    otile_v.at[p, h], out_hbm.at[t, :, itile0 + h], ssem.at[p]
                )

        def wait_store(t, itile0, p):
            for h in range(2):
                pltpu.make_async_copy(
                    otile_v.at[p, h], out_hbm.at[t, :, itile0 + h], ssem.at[p]
                ).wait()

        def per_sb(sb, carry):
            itile0 = (wid * sb_per_w + sb) * 2
            base = itile0 * IT * T
            pltpu.sync_copy(idx_hbm.at[pl.ds(base, SB * T)], ids_v)
            extract(0, 0)
            fire_gathers(0)

            def per_t2(t2, c):
                for b in range(2):
                    t = t2 * 2 + b
                    wait_gathers(b)

                    @pl.when(t + 1 < T)
                    def _(t=t, b=b):
                        extract(t + 1, 1 - b)
                        fire_gathers(1 - b)

                    @pl.when(t >= 2)
                    def _(t=t, b=b):
                        wait_store(t - 2, itile0, b)

                    transpose(b)
                    fire_store(t, itile0, b)
                return c

            lax.fori_loop(0, T // 2, per_t2, 0)
            wait_store(T - 2, itile0, 0)
            wait_store(T - 1, itile0, 1)
            return carry

        lax.fori_loop(0, sb_per_w, per_sb, 0)

    return gather_kernel


def kernel(token_ids, weight):
    Bt, T = token_ids.shape
    V, D = weight.shape
    idx = token_ids.reshape(Bt * T).astype(jnp.int32)
    n_full_slab_rows = (V // 128) * 128
    rem_flat = weight[n_full_slab_rows:, :].reshape(-1)
    table_flat = _build_detile(V, D)(weight.T, rem_flat)
    table = table_flat.reshape(V, D)
    flat5 = _build_gather(Bt, T, D)(idx, table)
    return flat5.transpose(2, 4, 0, 1, 3).reshape(Bt, T, D)


# speedup vs baseline: 2.4570x; 1.0914x over previous
import functools

import jax
import jax.numpy as jnp
from jax import lax
from jax.experimental import pallas as pl
from jax.experimental.pallas import tpu as pltpu
from jax.experimental.pallas import tpu_sc as plsc


@functools.lru_cache(maxsize=None)
def _build_detile(V, D):
    info = plsc.get_sparse_core_info()
    NC, NS = info.num_cores, info.num_subcores
    NW = NC * NS
    VT = 128
    n_full = V // VT
    rem = V % VT
    nj = n_full // NW
    n_extra = n_full % NW
    assert nj % 2 == 0 and rem % 16 == 0
    mesh = plsc.VectorSubcoreMesh(core_axis_name="c", subcore_axis_name="s")

    @functools.partial(
        pl.kernel,
        mesh=mesh,
        out_type=jax.ShapeDtypeStruct((V * D,), jnp.float32),
        scratch_types=[
            pltpu.VMEM((D, VT), jnp.float32),
            pltpu.VMEM((D, VT), jnp.float32),
            pltpu.VMEM((D * VT,), jnp.float32),
            pltpu.VMEM((D * VT,), jnp.float32),
            pltpu.SemaphoreType.DMA((2,)),
            pltpu.SemaphoreType.DMA((2,)),
        ],
        compiler_params=pltpu.CompilerParams(needs_layout_passes=False),
    )
    def detile_kernel(
        wt_hbm, rem_hbm, out_hbm, slab0, slab1, stage0, stage1, lsem, ssem
    ):
        wid = lax.axis_index("s") * NC + lax.axis_index("c")
        iota = lax.iota(jnp.int32, 16)
        row_vecs = [iota + vb * 16 for vb in range(VT // 16)]
        kbase = [(iota + k) & 15 for k in range(16)]
        slabs = (slab0, slab1)
        stages = (stage0, stage1)

        def vt_of(j):
            return j * NW + wid

        def fire_load(j, p):
            pltpu.async_copy(
                wt_hbm.at[:, pl.ds(vt_of(j) * VT, VT)], slabs[p], lsem.at[p]
            )

        def wait_load(j, p):
            pltpu.make_async_copy(
                wt_hbm.at[:, pl.ds(vt_of(j) * VT, VT)], slabs[p], lsem.at[p]
            ).wait()

        def transpose(p, n_vb):
            src = slabs[p]
            dst = stages[p]

            def per_db(db, c):
                for k in range(16):
                    dv = kbase[k] + db * 16
                    vals = [
                        plsc.load_gather(src, [dv, row_vecs[vb]])
                        for vb in range(n_vb)
                    ]
                    for vb in range(n_vb):
                        plsc.store_scatter(
                            dst, [row_vecs[vb] * D + dv], vals[vb]
                        )
                return c

            lax.fori_loop(0, D // 16, per_db, 0)

        def fire_store(j, p):
            pltpu.async_copy(
                stages[p], out_hbm.at[pl.ds(vt_of(j) * VT * D, VT * D)],
                ssem.at[p],
            )

        def wait_store(j, p):
            pltpu.make_async_copy(
                stages[p], out_hbm.at[pl.ds(vt_of(j) * VT * D, VT * D)],
                ssem.at[p],
            ).wait()

        fire_load(0, 0)
        fire_load(1, 1)

        def per_j2(j2, carry):
            for b in range(2):
                j = j2 * 2 + b
                wait_load(j, b)

                @pl.when(j >= 2)
                def _(j=j, b=b):
                    wait_store(j - 2, b)

                transpose(b, VT // 16)
                fire_store(j, b)

                @pl.when(j + 2 < nj)
                def _(j=j, b=b):
                    fire_load(j + 2, b)
            return carry

        lax.fori_loop(0, nj // 2, per_j2, 0)
        wait_store(nj - 2, 0)
        wait_store(nj - 1, 1)

        @pl.when(wid < n_extra)
        def _():
            vt = nj * NW + wid
            pltpu.sync_copy(wt_hbm.at[:, pl.ds(vt * VT, VT)], slab0)
            transpose(0, VT // 16)
            pltpu.sync_copy(stage0, out_hbm.at[pl.ds(vt * VT * D, VT * D)])

        if rem:

            @pl.when(wid == n_extra)
            def _():
                pltpu.sync_copy(rem_hbm, stage0.at[pl.ds(0, rem * D)])
                pltpu.sync_copy(
                    stage0.at[pl.ds(0, rem * D)],
                    out_hbm.at[pl.ds(n_full * VT * D, rem * D)],
                )

    return detile_kernel


@functools.lru_cache(maxsize=None)
def _build_gather(Bt, T, D):
    info = plsc.get_sparse_core_info()
    NC, NS = info.num_cores, info.num_subcores
    NW = NC * NS
    IT = 128
    SB = 2 * IT
    DT = D // 8
    n_itiles = Bt // IT
    sb_per_w = n_itiles // (2 * NW)
    assert Bt % IT == 0 and n_itiles % (2 * NW) == 0 and D % 16 == 0
    mesh = plsc.VectorSubcoreMesh(core_axis_name="c", subcore_axis_name="s")

    @functools.partial(
        pl.kernel,
        mesh=mesh,
        out_type=jax.ShapeDtypeStruct((T, DT, n_itiles, 8, IT), jnp.float32),
        scratch_types=[
            pltpu.VMEM((2, SB), jnp.int32),
            pltpu.VMEM((2, SB, D), jnp.float32),
            pltpu.VMEM((2, 2, DT, 8, IT), jnp.float32),
            pltpu.SemaphoreType.DMA((2,)),
            pltpu.SemaphoreType.DMA((2,)),
            pltpu.SemaphoreType.DMA((2,)),
        ],
        compiler_params=pltpu.CompilerParams(
            use_tc_tiling_on_sc=False, needs_layout_passes=False
        ),
    )
    def gather_kernel(
        idxt_hbm, table_hbm, out_hbm, idxcol_v, rows_v, otile_v, isem, gsem, ssem
    ):
        wid = lax.axis_index("s") * NC + lax.axis_index("c")
        iota = lax.iota(jnp.int32, 16)
        row_vecs = [iota + jb * 16 for jb in range(SB // 16)]
        kbase = [(iota + k) & 15 for k in range(16)]

        def fire_gathers(p):
            for h in range(2):
                pltpu.async_copy(
                    table_hbm.at[idxcol_v.at[p, pl.ds(h * IT, IT)]],
                    rows_v.at[p, pl.ds(h * IT, IT)],
                    gsem.at[p],
                )

        def wait_gathers(p):
            for h in range(2):
                pltpu.make_async_copy(
                    table_hbm.at[idxcol_v.at[p, pl.ds(h * IT, IT)]],
                    rows_v.at[p, pl.ds(h * IT, IT)],
                    gsem.at[p],
                ).wait()

        def transpose(pg, ps):
            src = rows_v.at[pg]

            def per_cb(cb, c):
                for k in range(16):
                    dv = kbase[k] + cb * 16
                    dtv = dv >> 3
                    dsv = dv & 7
                    vals = [
                        plsc.load_gather(src, [row_vecs[jb], dv])
                        for jb in range(SB // 16)
                    ]
                    for jb in range(SB // 16):
                        h = jb // (IT // 16)
                        plsc.store_scatter(
                            otile_v.at[ps, h],
                            [dtv, dsv, row_vecs[jb % (IT // 16)]],
                            vals[jb],
                        )
                return c

            lax.fori_loop(0, D // 16, per_cb, 0)

        def fire_store(t, itile0, p):
            for h in range(2):
                pltpu.async_copy(
                    otile_v.at[p, h], out_hbm.at[t, :, itile0 + h], ssem.at[p]
                )

        def wait_store(t, itile0, p):
            for h in range(2):
                pltpu.make_async_copy(
                    otile_v.at[p, h], out_hbm.at[t, :, itile0 + h], ssem.at[p]
                ).wait()

        def per_sb(sb, carry):
            itile0 = (wid * sb_per_w + sb) * 2
            i0 = itile0 * IT

            def fire_idx(t, q):
                pltpu.async_copy(
                    idxt_hbm.at[t, pl.ds(i0, SB)], idxcol_v.at[q], isem.at[q]
                )

            def wait_idx(t, q):
                pltpu.make_async_copy(
                    idxt_hbm.at[t, pl.ds(i0, SB)], idxcol_v.at[q], isem.at[q]
                ).wait()

            fire_idx(0, 0)
            fire_idx(1, 1)
            wait_idx(0, 0)
            fire_gathers(0)

            def step(t, pg):
                wait_gathers(pg)

                @pl.when(t + 2 < T)
                def _():
                    fire_idx(t + 2, pg)

                @pl.when(t + 1 < T)
                def _():
                    wait_idx(t + 1, 1 - pg)
                    fire_gathers(1 - pg)

                @pl.when(t >= 2)
                def _():
                    wait_store(t - 2, itile0, pg)

                transpose(pg, pg)
                fire_store(t, itile0, pg)

            def per_t2(t2, c):
                for b in range(2):
                    step(t2 * 2 + b, b)
                return c

            lax.fori_loop(0, T // 2, per_t2, 0)
            wait_store(T - 2, itile0, 0)
            wait_store(T - 1, itile0, 1)
            return carry

        lax.fori_loop(0, sb_per_w, per_sb, 0)

    return gather_kernel


def kernel(token_ids, weight):
    Bt, T = token_ids.shape
    V, D = weight.shape
    idxt = token_ids.astype(jnp.int32).T
    n_full_slab_rows = (V // 128) * 128
    rem_flat = weight[n_full_slab_rows:, :].reshape(-1)
    table_flat = _build_detile(V, D)(weight.T, rem_flat)
    table = table_flat.reshape(V, D)
    flat5 = _build_gather(Bt, T, D)(idxt, table)
    return flat5.transpose(2, 4, 0, 1, 3).reshape(Bt, T, D)
